# baseline (speedup 1.0000x reference)
import jax
import jax.numpy as jnp
from jax.experimental import pallas as pl

Q = 64
D = 64
K_TOTAL = 1_000_000
BLK = 8000
STEPS = K_TOTAL // BLK
THR = 0.4


def _top1_kernel(q_ref, k_ref, sim_ref, idx_ref):
    i = pl.program_id(0)

    @pl.when(i == 0)
    def _init():
        sim_ref[...] = jnp.full((Q, 1), -jnp.inf, jnp.float32)
        idx_ref[...] = jnp.zeros((Q, 1), jnp.int32)

    q = q_ref[...]
    qn = q / (jnp.sqrt(jnp.sum(q * q, axis=1, keepdims=True)) + 1e-9)
    k = k_ref[...]
    kn = k / (jnp.sqrt(jnp.sum(k * k, axis=1, keepdims=True)) + 1e-9)
    sims = jax.lax.dot_general(
        qn, kn, (((1,), (1,)), ((), ())), preferred_element_type=jnp.float32
    )
    m = jnp.max(sims, axis=1, keepdims=True)
    lane = jax.lax.broadcasted_iota(jnp.int32, sims.shape, 1)
    cand = jnp.where(sims == m, lane, jnp.int32(2**30))
    a = jnp.min(cand, axis=1, keepdims=True) + i * BLK

    best = sim_ref[...]
    improve = m > best
    sim_ref[...] = jnp.where(improve, m, best)
    idx_ref[...] = jnp.where(improve, a, idx_ref[...])


def kernel(queries, keys):
    sim, idx = pl.pallas_call(
        _top1_kernel,
        grid=(STEPS,),
        in_specs=[
            pl.BlockSpec((Q, D), lambda i: (0, 0)),
            pl.BlockSpec((BLK, D), lambda i: (i, 0)),
        ],
        out_specs=[
            pl.BlockSpec((Q, 1), lambda i: (0, 0)),
            pl.BlockSpec((Q, 1), lambda i: (0, 0)),
        ],
        out_shape=[
            jax.ShapeDtypeStruct((Q, 1), jnp.float32),
            jax.ShapeDtypeStruct((Q, 1), jnp.int32),
        ],
    )(queries, keys)
    best_sim = sim[:, 0]
    best_idx = idx[:, 0]
    valid = best_sim >= THR
    return best_sim, best_idx, valid
